# Initial kernel scaffold; baseline (speedup 1.0000x reference)
import jax
import jax.numpy as jnp
from jax.experimental import pallas as pl


def kernel(x, edge_index, edge_weight, W_xz, b_xz, W_hz, b_hz, W_xr, b_xr, W_hr, b_hr, W_xh, b_xh, W_hh, b_hh):
    raise NotImplementedError("write your pallas kernel here")


# speedup vs baseline: 26.8147x; 26.8147x over previous
import functools

import jax
import jax.numpy as jnp
from jax import lax
from jax.experimental import pallas as pl
from jax.experimental.pallas import tpu as pltpu
from jax.experimental.pallas import tpu_sc as plsc

N = 10000
E = 320000
D = 128
NC = 2
NS = 16
NW = NC * NS
L = 16

CH = 80
EDGES_DEG = E // NS
CH_DEG = EDGES_DEG // CH
EDGES_MAIN = E // NW
CH_MAIN = EDGES_MAIN // CH
NPAD = 10240
ROWS_TILE = NPAD // NS
G = 25
NB = CH_MAIN // G


def _rsqrt16(d16):
    i = lax.bitcast_convert_type(d16, jnp.int32)
    i = jnp.int32(0x5F3759DF) - lax.shift_right_arithmetic(i, 1)
    y = lax.bitcast_convert_type(i, jnp.float32)
    half = d16 * 0.5
    for _ in range(3):
        y = y * (1.5 - half * y * y)
    return jnp.where(d16 > 0.0, y, 0.0)


def _sc_body(x_hbm, row_t, ew_t, col_m, znd, znp,
             acc_out, dinv_out,
             acc_sp, deg_sp, row_blk, ew_blk, col_blk,
             rows_v, dinvT, sv, sem):
    c = lax.axis_index("c")
    s = lax.axis_index("s")

    pltpu.sync_copy(znd.at[pl.ds(s * ROWS_TILE, ROWS_TILE)],
                    acc_sp.at[pl.ds(s * ROWS_TILE, ROWS_TILE)])
    pltpu.sync_copy(znp.at[pl.ds(s * (NPAD // NS), NPAD // NS)],
                    deg_sp.at[pl.ds(s * (NPAD // NS), NPAD // NS)])

    plsc.subcore_barrier()

    for u in range(NC):
        for b in range(NB):
            pltpu.sync_copy(row_t.at[s, u, pl.ds(b * G, G)], row_blk)
            pltpu.sync_copy(ew_t.at[s, u, pl.ds(b * G, G)], ew_blk)

            @pl.loop(0, G)
            def _deg(j):
                pltpu.sync_copy(ew_blk.at[j], deg_sp.at[row_blk.at[j]],
                                add=True)

    plsc.subcore_barrier()

    pltpu.sync_copy(deg_sp, dinvT)

    @pl.loop(0, NPAD // L, unroll=4)
    def _rs(k):
        dinvT[pl.ds(k * L, L)] = _rsqrt16(dinvT[pl.ds(k * L, L)])

    for b in range(NB):
        pltpu.sync_copy(row_t.at[s, c, pl.ds(b * G, G)], row_blk)
        pltpu.sync_copy(ew_t.at[s, c, pl.ds(b * G, G)], ew_blk)
        pltpu.sync_copy(col_m.at[s, c, pl.ds(b * G, G)], col_blk)

        @pl.loop(0, G)
        def _main(j):
            gather = pltpu.async_copy(x_hbm.at[row_blk.at[j]], rows_v, sem)
            for t in range(CH // L):
                i16 = row_blk[j, pl.ds(t * L, L)]
                gd = plsc.load_gather(dinvT, [i16])
                sv[pl.ds(t * L, L)] = gd * ew_blk[j, pl.ds(t * L, L)]
            gather.wait()

            @pl.loop(0, CH // L)
            def _scale(g):
                s16 = sv[pl.ds(g * L, L)]
                base = g * L
                for ii in range(L):
                    f = s16[ii]
                    for q in range(D // L):
                        rows_v[base + ii, pl.ds(q * L, L)] = (
                            rows_v[base + ii, pl.ds(q * L, L)] * f)

            pltpu.sync_copy(rows_v, acc_sp.at[col_blk.at[j]], add=True)

    plsc.subcore_barrier()

    pltpu.sync_copy(acc_sp.at[pl.ds(s * ROWS_TILE, ROWS_TILE)],
                    acc_out.at[c, pl.ds(s * ROWS_TILE, ROWS_TILE)])

    @pl.when(s == 0)
    def _():
        pltpu.sync_copy(dinvT, dinv_out.at[c])


def _sc_scatter(x, row_t, ew_t, col_m, znd, znp):
    mesh = plsc.VectorSubcoreMesh(core_axis_name="c", subcore_axis_name="s")
    return pl.kernel(
        _sc_body,
        out_type=(
            jax.ShapeDtypeStruct((NC, NPAD, D), jnp.float32),
            jax.ShapeDtypeStruct((NC, NPAD), jnp.float32),
        ),
        mesh=mesh,
        compiler_params=pltpu.CompilerParams(
            needs_layout_passes=False, use_tc_tiling_on_sc=False),
        scratch_types=[
            pltpu.VMEM_SHARED((NPAD, D), jnp.float32),
            pltpu.VMEM_SHARED((NPAD,), jnp.float32),
            pltpu.VMEM((G, CH), jnp.int32),
            pltpu.VMEM((G, CH), jnp.float32),
            pltpu.VMEM((G, CH), jnp.int32),
            pltpu.VMEM((CH, D), jnp.float32),
            pltpu.VMEM((NPAD,), jnp.float32),
            pltpu.VMEM((CH,), jnp.float32),
            pltpu.SemaphoreType.DMA,
        ],
    )(x, row_t, ew_t, col_m, znd, znp)


BT = 1000


def _tc_body(x_ref, a0_ref, a1_ref, dv_ref, A0, A1, C0, C1, bz, bh, o_ref):
    dv = dv_ref[...]
    xb = x_ref[...]
    tx1 = -(dv * (a0_ref[...] + a1_ref[...]))
    z = jax.nn.sigmoid(
        jnp.dot(xb, A0[...], preferred_element_type=jnp.float32)
        + jnp.dot(tx1, A1[...], preferred_element_type=jnp.float32)
        + bz[...])
    ht = jnp.tanh(
        jnp.dot(xb, C0[...], preferred_element_type=jnp.float32)
        + jnp.dot(tx1, C1[...], preferred_element_type=jnp.float32)
        + bh[...])
    o_ref[...] = (1.0 - z) * ht


def _tc_finish(x, a0, a1, dinv, A0, A1, C0, C1, bz, bh):
    full = pl.BlockSpec((D, D), lambda i: (0, 0))
    vec = pl.BlockSpec((1, D), lambda i: (0, 0))
    return pl.pallas_call(
        _tc_body,
        grid=(N // BT,),
        in_specs=[
            pl.BlockSpec((BT, D), lambda i: (i, 0)),
            pl.BlockSpec((BT, D), lambda i: (i, 0)),
            pl.BlockSpec((BT, D), lambda i: (i, 0)),
            pl.BlockSpec((BT, 1), lambda i: (i, 0)),
            full, full, full, full, vec, vec,
        ],
        out_specs=pl.BlockSpec((BT, D), lambda i: (i, 0)),
        out_shape=jax.ShapeDtypeStruct((N, D), jnp.float32),
    )(x, a0, a1, dinv, A0, A1, C0, C1, bz, bh)


def kernel(x, edge_index, edge_weight,
           W_xz, b_xz, W_hz, b_hz, W_xr, b_xr, W_hr, b_hr,
           W_xh, b_xh, W_hh, b_hh):
    row = edge_index[0]
    col = edge_index[1]
    row_t = row.reshape(NS, NC, CH_MAIN, CH)
    ew_t = edge_weight.reshape(NS, NC, CH_MAIN, CH)
    col_m = col.reshape(NS, NC, CH_MAIN, CH)
    znd = jnp.zeros((NPAD, D), jnp.float32)
    znp = jnp.zeros((NPAD,), jnp.float32)

    acc, dinv = _sc_scatter(x, row_t, ew_t, col_m, znd, znp)

    return _tc_finish(
        x, acc[0], acc[1], dinv[0, :N].reshape(N, 1),
        W_xz[0], W_xz[1], W_xh[0], W_xh[1],
        (b_xz + b_hz).reshape(1, D), (b_xh + b_hh).reshape(1, D),
    )


# speedup vs baseline: 28.0705x; 1.0468x over previous
import jax
import jax.numpy as jnp
from jax import lax
from jax.experimental import pallas as pl
from jax.experimental.pallas import tpu as pltpu
from jax.experimental.pallas import tpu_sc as plsc

N = 10000
E = 320000
D = 128
NC = 2
NS = 16
NW = NC * NS
L = 16

CH = 64
EDGES_MAIN = E // NW
EPAD = 10048
CH_MAIN = EPAD // CH
NPAD = 10240
ROWS_ACC = N // NS
G = 16
DEG_BLOCKS = [(b * G, min(G, CH_MAIN - b * G))
              for b in range((CH_MAIN + G - 1) // G)]


def _rsqrt16(d16):
    i = lax.bitcast_convert_type(d16, jnp.int32)
    i = jnp.int32(0x5F3759DF) - lax.shift_right_arithmetic(i, 1)
    y = lax.bitcast_convert_type(i, jnp.float32)
    half = d16 * 0.5
    for _ in range(3):
        y = y * (1.5 - half * y * y)
    return jnp.where(d16 > 0.0, y, 0.0)


def _sc_body(x_hbm, row_t, ew_t, col_m, znd, znp,
             acc_out, dinv_out,
             acc_sp, deg_sp, row_mv, ew_mv, col_blk, row_blk, ew_blk,
             rows0, rows1, dinvT, g0, g1, s0, s1, dsem):
    c = lax.axis_index("c")
    s = lax.axis_index("s")

    pltpu.sync_copy(znd.at[pl.ds(s * ROWS_ACC, ROWS_ACC)],
                    acc_sp.at[pl.ds(s * ROWS_ACC, ROWS_ACC)])
    pltpu.sync_copy(znp.at[pl.ds(s * (NPAD // NS), NPAD // NS)],
                    deg_sp.at[pl.ds(s * (NPAD // NS), NPAD // NS)])

    pltpu.sync_copy(row_t.at[s, c], row_mv)
    pltpu.sync_copy(ew_t.at[s, c], ew_mv)

    plsc.subcore_barrier()

    for u in range(NC):
        for start, cnt in DEG_BLOCKS:
            pltpu.sync_copy(row_t.at[s, u, pl.ds(start, cnt)],
                            row_blk.at[pl.ds(0, cnt)])
            pltpu.sync_copy(ew_t.at[s, u, pl.ds(start, cnt)],
                            ew_blk.at[pl.ds(0, cnt)])

            @pl.loop(0, cnt)
            def _deg(j):
                pltpu.async_copy(ew_blk.at[j], deg_sp.at[row_blk.at[j]], dsem,
                                 add=True)

            pltpu.make_async_copy(ew_t.at[s, u, pl.ds(start, cnt)],
                                  ew_blk.at[pl.ds(0, cnt)], dsem).wait()

    plsc.subcore_barrier()

    pltpu.sync_copy(deg_sp.at[pl.ds(0, N)], dinvT)

    @pl.loop(0, N // L, unroll=4)
    def _rs(k):
        dinvT[pl.ds(k * L, L)] = _rsqrt16(dinvT[pl.ds(k * L, L)])

    @pl.loop(0, CH_MAIN, unroll=2)
    def _pre(j):
        for t in range(CH // L):
            i16 = row_mv[j, pl.ds(t * L, L)]
            gd = plsc.load_gather(dinvT, [i16])
            ew_mv[j, pl.ds(t * L, L)] = gd * ew_mv[j, pl.ds(t * L, L)]

    rows = (rows0, rows1)
    gsem = (g0, g1)
    ssem = (s0, s1)

    def _chunk(j, j_local, cnt, bufp, gp, sp, bufq, gq, sq):
        pltpu.make_async_copy(x_hbm.at[pl.ds(0, CH)], bufp, gp).wait()

        @pl.when(j_local > 0)
        def _():
            pltpu.make_async_copy(x_hbm.at[pl.ds(0, CH)], bufq, sq).wait()

        @pl.when(j_local < cnt - 1)
        def _():
            pltpu.async_copy(x_hbm.at[row_mv.at[j + 1]], bufq, gq)

        @pl.loop(0, CH // L)
        def _scale(g):
            s16 = ew_mv[j, pl.ds(g * L, L)]
            base = g * L
            for ii in range(L):
                f = s16[ii]
                for q in range(D // L):
                    bufp[base + ii, pl.ds(q * L, L)] = (
                        bufp[base + ii, pl.ds(q * L, L)] * f)

        pltpu.async_copy(bufp, acc_sp.at[col_blk.at[j_local]], sp, add=True)

    for start, cnt in DEG_BLOCKS:
        if start > 0:
            pp = (start - 1) % 2
            pltpu.make_async_copy(x_hbm.at[pl.ds(0, CH)], rows[pp],
                                  ssem[pp]).wait()
        pltpu.sync_copy(col_m.at[s, c, pl.ds(start, cnt)],
                        col_blk.at[pl.ds(0, cnt)])
        p0 = start % 2
        pltpu.async_copy(x_hbm.at[row_mv.at[start]], rows[p0], gsem[p0])

        @pl.loop(0, cnt)
        def _main(j_local):
            j = start + j_local

            @pl.when(j % 2 == 0)
            def _():
                _chunk(j, j_local, cnt, rows0, g0, s0, rows1, g1, s1)

            @pl.when(j % 2 == 1)
            def _():
                _chunk(j, j_local, cnt, rows1, g1, s1, rows0, g0, s0)

    pltpu.make_async_copy(x_hbm.at[pl.ds(0, CH)], rows0, s0).wait()

    plsc.subcore_barrier()

    pltpu.sync_copy(acc_sp.at[pl.ds(s * ROWS_ACC, ROWS_ACC)],
                    acc_out.at[c, pl.ds(s * ROWS_ACC, ROWS_ACC)])

    @pl.when(s == 0)
    def _():
        pltpu.sync_copy(dinvT, dinv_out.at[c])


def _sc_scatter(x, row_t, ew_t, col_m, znd, znp):
    mesh = plsc.VectorSubcoreMesh(core_axis_name="c", subcore_axis_name="s")
    return pl.kernel(
        _sc_body,
        out_type=(
            jax.ShapeDtypeStruct((NC, N, D), jnp.float32),
            jax.ShapeDtypeStruct((NC, N), jnp.float32),
        ),
        mesh=mesh,
        compiler_params=pltpu.CompilerParams(
            needs_layout_passes=False, use_tc_tiling_on_sc=False),
        scratch_types=[
            pltpu.VMEM_SHARED((N, D), jnp.float32),
            pltpu.VMEM_SHARED((NPAD,), jnp.float32),
            pltpu.VMEM((CH_MAIN, CH), jnp.int32),
            pltpu.VMEM((CH_MAIN, CH), jnp.float32),
            pltpu.VMEM((G, CH), jnp.int32),
            pltpu.VMEM((G, CH), jnp.int32),
            pltpu.VMEM((G, CH), jnp.float32),
            pltpu.VMEM((CH, D), jnp.float32),
            pltpu.VMEM((CH, D), jnp.float32),
            pltpu.VMEM((N,), jnp.float32),
            pltpu.SemaphoreType.DMA,
            pltpu.SemaphoreType.DMA,
            pltpu.SemaphoreType.DMA,
            pltpu.SemaphoreType.DMA,
            pltpu.SemaphoreType.DMA,
        ],
    )(x, row_t, ew_t, col_m, znd, znp)


BT = 1000


def _tc_body(x_ref, a0_ref, a1_ref, dv_ref, A0, A1, C0, C1, bz, bh, o_ref):
    dv = dv_ref[...]
    xb = x_ref[...]
    tx1 = -(dv * (a0_ref[...] + a1_ref[...]))
    z = jax.nn.sigmoid(
        jnp.dot(xb, A0[...], preferred_element_type=jnp.float32)
        + jnp.dot(tx1, A1[...], preferred_element_type=jnp.float32)
        + bz[...])
    ht = jnp.tanh(
        jnp.dot(xb, C0[...], preferred_element_type=jnp.float32)
        + jnp.dot(tx1, C1[...], preferred_element_type=jnp.float32)
        + bh[...])
    o_ref[...] = (1.0 - z) * ht


def _tc_finish(x, a0, a1, dinv, A0, A1, C0, C1, bz, bh):
    full = pl.BlockSpec((D, D), lambda i: (0, 0))
    vec = pl.BlockSpec((1, D), lambda i: (0, 0))
    return pl.pallas_call(
        _tc_body,
        grid=(N // BT,),
        in_specs=[
            pl.BlockSpec((BT, D), lambda i: (i, 0)),
            pl.BlockSpec((BT, D), lambda i: (i, 0)),
            pl.BlockSpec((BT, D), lambda i: (i, 0)),
            pl.BlockSpec((BT, 1), lambda i: (i, 0)),
            full, full, full, full, vec, vec,
        ],
        out_specs=pl.BlockSpec((BT, D), lambda i: (i, 0)),
        out_shape=jax.ShapeDtypeStruct((N, D), jnp.float32),
    )(x, a0, a1, dinv, A0, A1, C0, C1, bz, bh)


def kernel(x, edge_index, edge_weight,
           W_xz, b_xz, W_hz, b_hz, W_xr, b_xr, W_hr, b_hr,
           W_xh, b_xh, W_hh, b_hh):
    row = edge_index[0]
    col = edge_index[1]
    pad_i = jnp.zeros((NW, EPAD - EDGES_MAIN), jnp.int32)
    pad_f = jnp.zeros((NW, EPAD - EDGES_MAIN), jnp.float32)
    row_t = jnp.concatenate([row.reshape(NW, EDGES_MAIN), pad_i], axis=1
                            ).reshape(NS, NC, CH_MAIN, CH)
    ew_t = jnp.concatenate([edge_weight.reshape(NW, EDGES_MAIN), pad_f], axis=1
                           ).reshape(NS, NC, CH_MAIN, CH)
    col_m = jnp.concatenate([col.reshape(NW, EDGES_MAIN), pad_i], axis=1
                            ).reshape(NS, NC, CH_MAIN, CH)
    znd = jnp.zeros((N, D), jnp.float32)
    znp = jnp.zeros((NPAD,), jnp.float32)

    acc, dinv = _sc_scatter(x, row_t, ew_t, col_m, znd, znp)

    return _tc_finish(
        x, acc[0], acc[1], dinv[0].reshape(N, 1),
        W_xz[0], W_xz[1], W_xh[0], W_xh[1],
        (b_xz + b_hz).reshape(1, D), (b_xh + b_hh).reshape(1, D),
    )


# speedup vs baseline: 28.0806x; 1.0004x over previous
import jax
import jax.numpy as jnp
from jax import lax
from jax.experimental import pallas as pl
from jax.experimental.pallas import tpu as pltpu
from jax.experimental.pallas import tpu_sc as plsc

N = 10000
E = 320000
D = 128
NC = 2
NS = 16
NW = NC * NS
L = 16

CH = 64
EDGES_MAIN = E // NW
EPAD = 10048
CH_MAIN = EPAD // CH
NPAD = 10240
ROWS_ACC = N // NS
G = 16
DEG_BLOCKS = [(b * G, min(G, CH_MAIN - b * G))
              for b in range((CH_MAIN + G - 1) // G)]


def _rsqrt16(d16):
    i = lax.bitcast_convert_type(d16, jnp.int32)
    i = jnp.int32(0x5F3759DF) - lax.shift_right_arithmetic(i, 1)
    y = lax.bitcast_convert_type(i, jnp.float32)
    half = d16 * 0.5
    for _ in range(3):
        y = y * (1.5 - half * y * y)
    return jnp.where(d16 > 0.0, y, 0.0)


def _sc_body(x_hbm, row_t, ew_t, col_m, znd, znp,
             acc_out, dinv_out,
             acc_sp, deg_sp, row_mv, ew_mv, col_blk, row_blk, ew_blk,
             rows0, rows1, dinvT, g0, g1, s0, s1, dsem):
    c = lax.axis_index("c")
    s = lax.axis_index("s")

    pltpu.sync_copy(znd.at[pl.ds(s * ROWS_ACC, ROWS_ACC)],
                    acc_sp.at[pl.ds(s * ROWS_ACC, ROWS_ACC)])
    pltpu.sync_copy(znp.at[pl.ds(s * (NPAD // NS), NPAD // NS)],
                    deg_sp.at[pl.ds(s * (NPAD // NS), NPAD // NS)])

    pltpu.sync_copy(row_t.at[s, c], row_mv)
    pltpu.sync_copy(ew_t.at[s, c], ew_mv)

    plsc.subcore_barrier()

    def _deg_phase():
      for u in range(NC):
        for start, cnt in DEG_BLOCKS:
            pltpu.sync_copy(row_t.at[s, u, pl.ds(start, cnt)],
                            row_blk.at[pl.ds(0, cnt)])
            pltpu.sync_copy(ew_t.at[s, u, pl.ds(start, cnt)],
                            ew_blk.at[pl.ds(0, cnt)])

            @pl.loop(0, cnt)
            def _deg(j):
                pltpu.async_copy(ew_blk.at[j], deg_sp.at[row_blk.at[j]], dsem,
                                 add=True)

            pltpu.make_async_copy(ew_t.at[s, u, pl.ds(start, cnt)],
                                  ew_blk.at[pl.ds(0, cnt)], dsem).wait()

    with jax.named_scope("deg_phase"):
        _deg_phase()
    plsc.subcore_barrier()

    def _dinv_phase():
      pltpu.sync_copy(deg_sp.at[pl.ds(0, N)], dinvT)

      @pl.loop(0, N // L, unroll=4)
      def _rs(k):
          dinvT[pl.ds(k * L, L)] = _rsqrt16(dinvT[pl.ds(k * L, L)])

      @pl.loop(0, CH_MAIN, unroll=2)
      def _pre(j):
          for t in range(CH // L):
              i16 = row_mv[j, pl.ds(t * L, L)]
              gd = plsc.load_gather(dinvT, [i16])
              ew_mv[j, pl.ds(t * L, L)] = gd * ew_mv[j, pl.ds(t * L, L)]

    with jax.named_scope("dinv_phase"):
        _dinv_phase()
    rows = (rows0, rows1)
    gsem = (g0, g1)
    ssem = (s0, s1)

    def _chunk(j, j_local, cnt, bufp, gp, sp, bufq, gq, sq):
        pltpu.make_async_copy(x_hbm.at[pl.ds(0, CH)], bufp, gp).wait()

        @pl.when(j_local > 0)
        def _():
            pltpu.make_async_copy(x_hbm.at[pl.ds(0, CH)], bufq, sq).wait()

        @pl.when(j_local < cnt - 1)
        def _():
            pltpu.async_copy(x_hbm.at[row_mv.at[j + 1]], bufq, gq)

        @pl.loop(0, CH // L)
        def _scale(g):
            s16 = ew_mv[j, pl.ds(g * L, L)]
            base = g * L
            for ii in range(L):
                f = s16[ii]
                for q in range(D // L):
                    bufp[base + ii, pl.ds(q * L, L)] = (
                        bufp[base + ii, pl.ds(q * L, L)] * f)

        pltpu.async_copy(bufp, acc_sp.at[col_blk.at[j_local]], sp, add=True)

    def _main_phase():
      for start, cnt in DEG_BLOCKS:
        if start > 0:
            pp = (start - 1) % 2
            pltpu.make_async_copy(x_hbm.at[pl.ds(0, CH)], rows[pp],
                                  ssem[pp]).wait()
        pltpu.sync_copy(col_m.at[s, c, pl.ds(start, cnt)],
                        col_blk.at[pl.ds(0, cnt)])
        p0 = start % 2
        pltpu.async_copy(x_hbm.at[row_mv.at[start]], rows[p0], gsem[p0])

        @pl.loop(0, cnt)
        def _main(j_local):
            j = start + j_local

            @pl.when(j % 2 == 0)
            def _():
                _chunk(j, j_local, cnt, rows0, g0, s0, rows1, g1, s1)

            @pl.when(j % 2 == 1)
            def _():
                _chunk(j, j_local, cnt, rows1, g1, s1, rows0, g0, s0)

      pltpu.make_async_copy(x_hbm.at[pl.ds(0, CH)], rows0, s0).wait()

    with jax.named_scope("main_phase"):
        _main_phase()
    plsc.subcore_barrier()

    pltpu.sync_copy(acc_sp.at[pl.ds(s * ROWS_ACC, ROWS_ACC)],
                    acc_out.at[c, pl.ds(s * ROWS_ACC, ROWS_ACC)])

    @pl.when(s == 0)
    def _():
        pltpu.sync_copy(dinvT, dinv_out.at[c])


def _sc_scatter(x, row_t, ew_t, col_m, znd, znp):
    mesh = plsc.VectorSubcoreMesh(core_axis_name="c", subcore_axis_name="s")
    return pl.kernel(
        _sc_body,
        out_type=(
            jax.ShapeDtypeStruct((NC, N, D), jnp.float32),
            jax.ShapeDtypeStruct((NC, N), jnp.float32),
        ),
        mesh=mesh,
        compiler_params=pltpu.CompilerParams(
            needs_layout_passes=False, use_tc_tiling_on_sc=False),
        scratch_types=[
            pltpu.VMEM_SHARED((N, D), jnp.float32),
            pltpu.VMEM_SHARED((NPAD,), jnp.float32),
            pltpu.VMEM((CH_MAIN, CH), jnp.int32),
            pltpu.VMEM((CH_MAIN, CH), jnp.float32),
            pltpu.VMEM((G, CH), jnp.int32),
            pltpu.VMEM((G, CH), jnp.int32),
            pltpu.VMEM((G, CH), jnp.float32),
            pltpu.VMEM((CH, D), jnp.float32),
            pltpu.VMEM((CH, D), jnp.float32),
            pltpu.VMEM((N,), jnp.float32),
            pltpu.SemaphoreType.DMA,
            pltpu.SemaphoreType.DMA,
            pltpu.SemaphoreType.DMA,
            pltpu.SemaphoreType.DMA,
            pltpu.SemaphoreType.DMA,
        ],
    )(x, row_t, ew_t, col_m, znd, znp)


BT = 1000


def _tc_body(x_ref, a0_ref, a1_ref, dv_ref, A0, A1, C0, C1, bz, bh, o_ref):
    dv = dv_ref[...]
    xb = x_ref[...]
    tx1 = -(dv * (a0_ref[...] + a1_ref[...]))
    z = jax.nn.sigmoid(
        jnp.dot(xb, A0[...], preferred_element_type=jnp.float32)
        + jnp.dot(tx1, A1[...], preferred_element_type=jnp.float32)
        + bz[...])
    ht = jnp.tanh(
        jnp.dot(xb, C0[...], preferred_element_type=jnp.float32)
        + jnp.dot(tx1, C1[...], preferred_element_type=jnp.float32)
        + bh[...])
    o_ref[...] = (1.0 - z) * ht


def _tc_finish(x, a0, a1, dinv, A0, A1, C0, C1, bz, bh):
    full = pl.BlockSpec((D, D), lambda i: (0, 0))
    vec = pl.BlockSpec((1, D), lambda i: (0, 0))
    return pl.pallas_call(
        _tc_body,
        grid=(N // BT,),
        in_specs=[
            pl.BlockSpec((BT, D), lambda i: (i, 0)),
            pl.BlockSpec((BT, D), lambda i: (i, 0)),
            pl.BlockSpec((BT, D), lambda i: (i, 0)),
            pl.BlockSpec((BT, 1), lambda i: (i, 0)),
            full, full, full, full, vec, vec,
        ],
        out_specs=pl.BlockSpec((BT, D), lambda i: (i, 0)),
        out_shape=jax.ShapeDtypeStruct((N, D), jnp.float32),
    )(x, a0, a1, dinv, A0, A1, C0, C1, bz, bh)


def kernel(x, edge_index, edge_weight,
           W_xz, b_xz, W_hz, b_hz, W_xr, b_xr, W_hr, b_hr,
           W_xh, b_xh, W_hh, b_hh):
    row = edge_index[0]
    col = edge_index[1]
    pad_i = jnp.zeros((NW, EPAD - EDGES_MAIN), jnp.int32)
    pad_f = jnp.zeros((NW, EPAD - EDGES_MAIN), jnp.float32)
    row_t = jnp.concatenate([row.reshape(NW, EDGES_MAIN), pad_i], axis=1
                            ).reshape(NS, NC, CH_MAIN, CH)
    ew_t = jnp.concatenate([edge_weight.reshape(NW, EDGES_MAIN), pad_f], axis=1
                           ).reshape(NS, NC, CH_MAIN, CH)
    col_m = jnp.concatenate([col.reshape(NW, EDGES_MAIN), pad_i], axis=1
                            ).reshape(NS, NC, CH_MAIN, CH)
    znd = jnp.zeros((N, D), jnp.float32)
    znp = jnp.zeros((NPAD,), jnp.float32)

    acc, dinv = _sc_scatter(x, row_t, ew_t, col_m, znd, znp)

    return _tc_finish(
        x, acc[0], acc[1], dinv[0].reshape(N, 1),
        W_xz[0], W_xz[1], W_xh[0], W_xh[1],
        (b_xz + b_hz).reshape(1, D), (b_xh + b_hh).reshape(1, D),
    )


# speedup vs baseline: 31.3889x; 1.1178x over previous
import jax
import jax.numpy as jnp
from jax import lax
from jax.experimental import pallas as pl
from jax.experimental.pallas import tpu as pltpu
from jax.experimental.pallas import tpu_sc as plsc

N = 10000
E = 320000
D = 128
NC = 2
NS = 16
NW = NC * NS
L = 16

CH = 64
EDGES_MAIN = E // NW
EPAD = 10048
CH_MAIN = EPAD // CH
NPAD = 10240
ROWS_ACC = N // NS
G = 16
DEG_BLOCKS = [(b * G, min(G, CH_MAIN - b * G))
              for b in range((CH_MAIN + G - 1) // G)]


def _rsqrt16(d16):
    i = lax.bitcast_convert_type(d16, jnp.int32)
    i = jnp.int32(0x5F3759DF) - lax.shift_right_arithmetic(i, 1)
    y = lax.bitcast_convert_type(i, jnp.float32)
    half = d16 * 0.5
    for _ in range(3):
        y = y * (1.5 - half * y * y)
    return jnp.where(d16 > 0.0, y, 0.0)


def _sc_body(x_hbm, row_t, ew_t, col_m, znd, znp,
             acc_out, dinv_out,
             acc_sp, deg_sp, row_mv, col_blk, row_blk, ew_blk,
             rows0, rows1, rows2, dinvT, g0, g1, g2, s0, s1, s2, dsem):
    c = lax.axis_index("c")
    s = lax.axis_index("s")

    pltpu.sync_copy(znd.at[pl.ds(s * ROWS_ACC, ROWS_ACC)],
                    acc_sp.at[pl.ds(s * ROWS_ACC, ROWS_ACC)])
    pltpu.sync_copy(znp.at[pl.ds(s * (NPAD // NS), NPAD // NS)],
                    deg_sp.at[pl.ds(s * (NPAD // NS), NPAD // NS)])

    pltpu.sync_copy(row_t.at[s, c], row_mv)

    plsc.subcore_barrier()

    def _deg_phase():
      for u in range(NC):
        for start, cnt in DEG_BLOCKS:
            pltpu.sync_copy(row_t.at[s, u, pl.ds(start, cnt)],
                            row_blk.at[pl.ds(0, cnt)])
            pltpu.sync_copy(ew_t.at[s, u, pl.ds(start, cnt)],
                            ew_blk.at[pl.ds(0, cnt)])

            @pl.loop(0, cnt)
            def _deg(j):
                pltpu.async_copy(ew_blk.at[j], deg_sp.at[row_blk.at[j]], dsem,
                                 add=True)

            pltpu.make_async_copy(ew_t.at[s, u, pl.ds(start, cnt)],
                                  ew_blk.at[pl.ds(0, cnt)], dsem).wait()

    with jax.named_scope("deg_phase"):
        _deg_phase()
    plsc.subcore_barrier()

    def _dinv_phase():
      pltpu.sync_copy(deg_sp.at[pl.ds(0, N)], dinvT)

      @pl.loop(0, N // L, unroll=4)
      def _rs(k):
          dinvT[pl.ds(k * L, L)] = _rsqrt16(dinvT[pl.ds(k * L, L)])

    with jax.named_scope("dinv_phase"):
        _dinv_phase()
    rows = (rows0, rows1, rows2)
    gsem = (g0, g1, g2)
    ssem = (s0, s1, s2)
    NBUF = 3

    def _chunk(j, j_local, cnt, p):
        bufp, gp, sp = rows[p], gsem[p], ssem[p]
        q = (p + 2) % NBUF
        pltpu.make_async_copy(x_hbm.at[pl.ds(0, CH)], bufp, gp).wait()

        @pl.when(j_local > 0)
        def _():
            pltpu.make_async_copy(x_hbm.at[pl.ds(0, CH)], rows[q],
                                  ssem[q]).wait()

        @pl.when(j_local < cnt - 2)
        def _():
            pltpu.async_copy(x_hbm.at[row_mv.at[j + 2]], rows[q], gsem[q])

        @pl.loop(0, CH // L)
        def _scale(g):
            i16 = row_mv[j, pl.ds(g * L, L)]
            s16 = (plsc.load_gather(dinvT, [i16])
                   * ew_blk[j_local, pl.ds(g * L, L)])
            base = g * L
            for ii in range(L):
                f = s16[ii]
                for qq in range(D // L):
                    bufp[base + ii, pl.ds(qq * L, L)] = (
                        bufp[base + ii, pl.ds(qq * L, L)] * f)

        pltpu.async_copy(bufp, acc_sp.at[col_blk.at[j_local]], sp, add=True)

    def _main_phase():
      for start, cnt in DEG_BLOCKS:
        if start > 0:
            pp = (start - 1) % NBUF
            pltpu.make_async_copy(x_hbm.at[pl.ds(0, CH)], rows[pp],
                                  ssem[pp]).wait()
        pltpu.sync_copy(col_m.at[s, c, pl.ds(start, cnt)],
                        col_blk.at[pl.ds(0, cnt)])
        pltpu.sync_copy(ew_t.at[s, c, pl.ds(start, cnt)],
                        ew_blk.at[pl.ds(0, cnt)])
        pltpu.async_copy(x_hbm.at[row_mv.at[start]],
                         rows[start % NBUF], gsem[start % NBUF])
        pltpu.async_copy(x_hbm.at[row_mv.at[start + 1]],
                         rows[(start + 1) % NBUF], gsem[(start + 1) % NBUF])

        @pl.loop(0, cnt)
        def _main(j_local):
            j = start + j_local

            for pv in range(NBUF):
                @pl.when(j % NBUF == pv)
                def _(pv=pv):
                    _chunk(j, j_local, cnt, pv)

      pltpu.make_async_copy(x_hbm.at[pl.ds(0, CH)], rows0, s0).wait()

    with jax.named_scope("main_phase"):
        _main_phase()
    plsc.subcore_barrier()

    pltpu.sync_copy(acc_sp.at[pl.ds(s * ROWS_ACC, ROWS_ACC)],
                    acc_out.at[c, pl.ds(s * ROWS_ACC, ROWS_ACC)])

    @pl.when(s == 0)
    def _():
        pltpu.sync_copy(dinvT, dinv_out.at[c])


def _sc_scatter(x, row_t, ew_t, col_m, znd, znp):
    mesh = plsc.VectorSubcoreMesh(core_axis_name="c", subcore_axis_name="s")
    return pl.kernel(
        _sc_body,
        out_type=(
            jax.ShapeDtypeStruct((NC, N, D), jnp.float32),
            jax.ShapeDtypeStruct((NC, N), jnp.float32),
        ),
        mesh=mesh,
        compiler_params=pltpu.CompilerParams(
            needs_layout_passes=False, use_tc_tiling_on_sc=False),
        scratch_types=[
            pltpu.VMEM_SHARED((N, D), jnp.float32),
            pltpu.VMEM_SHARED((NPAD,), jnp.float32),
            pltpu.VMEM((CH_MAIN, CH), jnp.int32),
            pltpu.VMEM((G, CH), jnp.int32),
            pltpu.VMEM((G, CH), jnp.int32),
            pltpu.VMEM((G, CH), jnp.float32),
            pltpu.VMEM((CH, D), jnp.float32),
            pltpu.VMEM((CH, D), jnp.float32),
            pltpu.VMEM((CH, D), jnp.float32),
            pltpu.VMEM((N,), jnp.float32),
            pltpu.SemaphoreType.DMA,
            pltpu.SemaphoreType.DMA,
            pltpu.SemaphoreType.DMA,
            pltpu.SemaphoreType.DMA,
            pltpu.SemaphoreType.DMA,
            pltpu.SemaphoreType.DMA,
            pltpu.SemaphoreType.DMA,
        ],
    )(x, row_t, ew_t, col_m, znd, znp)


BT = 1000


def _tc_body(x_ref, a0_ref, a1_ref, dv_ref, A0, A1, C0, C1, bz, bh, o_ref):
    dv = dv_ref[...]
    xb = x_ref[...]
    tx1 = -(dv * (a0_ref[...] + a1_ref[...]))
    z = jax.nn.sigmoid(
        jnp.dot(xb, A0[...], preferred_element_type=jnp.float32)
        + jnp.dot(tx1, A1[...], preferred_element_type=jnp.float32)
        + bz[...])
    ht = jnp.tanh(
        jnp.dot(xb, C0[...], preferred_element_type=jnp.float32)
        + jnp.dot(tx1, C1[...], preferred_element_type=jnp.float32)
        + bh[...])
    o_ref[...] = (1.0 - z) * ht


def _tc_finish(x, a0, a1, dinv, A0, A1, C0, C1, bz, bh):
    full = pl.BlockSpec((D, D), lambda i: (0, 0))
    vec = pl.BlockSpec((1, D), lambda i: (0, 0))
    return pl.pallas_call(
        _tc_body,
        grid=(N // BT,),
        in_specs=[
            pl.BlockSpec((BT, D), lambda i: (i, 0)),
            pl.BlockSpec((BT, D), lambda i: (i, 0)),
            pl.BlockSpec((BT, D), lambda i: (i, 0)),
            pl.BlockSpec((BT, 1), lambda i: (i, 0)),
            full, full, full, full, vec, vec,
        ],
        out_specs=pl.BlockSpec((BT, D), lambda i: (i, 0)),
        out_shape=jax.ShapeDtypeStruct((N, D), jnp.float32),
    )(x, a0, a1, dinv, A0, A1, C0, C1, bz, bh)


def kernel(x, edge_index, edge_weight,
           W_xz, b_xz, W_hz, b_hz, W_xr, b_xr, W_hr, b_hr,
           W_xh, b_xh, W_hh, b_hh):
    row = edge_index[0]
    col = edge_index[1]
    pad_i = jnp.zeros((NW, EPAD - EDGES_MAIN), jnp.int32)
    pad_f = jnp.zeros((NW, EPAD - EDGES_MAIN), jnp.float32)
    row_t = jnp.concatenate([row.reshape(NW, EDGES_MAIN), pad_i], axis=1
                            ).reshape(NS, NC, CH_MAIN, CH)
    ew_t = jnp.concatenate([edge_weight.reshape(NW, EDGES_MAIN), pad_f], axis=1
                           ).reshape(NS, NC, CH_MAIN, CH)
    col_m = jnp.concatenate([col.reshape(NW, EDGES_MAIN), pad_i], axis=1
                            ).reshape(NS, NC, CH_MAIN, CH)
    znd = jnp.zeros((N, D), jnp.float32)
    znp = jnp.zeros((NPAD,), jnp.float32)

    acc, dinv = _sc_scatter(x, row_t, ew_t, col_m, znd, znp)

    return _tc_finish(
        x, acc[0], acc[1], dinv[0].reshape(N, 1),
        W_xz[0], W_xz[1], W_xh[0], W_xh[1],
        (b_xz + b_hz).reshape(1, D), (b_xh + b_hh).reshape(1, D),
    )


# speedup vs baseline: 33.3317x; 1.0619x over previous
import jax
import jax.numpy as jnp
from jax import lax
from jax.experimental import pallas as pl
from jax.experimental.pallas import tpu as pltpu
from jax.experimental.pallas import tpu_sc as plsc

N = 10000
E = 320000
D = 128
NC = 2
NS = 16
NW = NC * NS
L = 16

CH = 64
EDGES_MAIN = E // NW
EPAD = 10048
CH_MAIN = EPAD // CH
NPAD = 10240
ROWS_ACC = N // NS
G = 16
DEG_BLOCKS = [(b * G, min(G, CH_MAIN - b * G))
              for b in range((CH_MAIN + G - 1) // G)]


def _rsqrt16(d16):
    i = lax.bitcast_convert_type(d16, jnp.int32)
    i = jnp.int32(0x5F3759DF) - lax.shift_right_arithmetic(i, 1)
    y = lax.bitcast_convert_type(i, jnp.float32)
    half = d16 * 0.5
    for _ in range(3):
        y = y * (1.5 - half * y * y)
    return jnp.where(d16 > 0.0, y, 0.0)


def _sc_body(x_hbm, row_t, ew_t, col_m, znd, znp,
             acc_out, dinv_out,
             acc_sp, deg_sp, row_h0, row_h1, col_blk, row_blk, ew_blk,
             rows0, rows1, rows2, rows3, dinvT, g0, g1, g2, g3,
             s0, s1, s2, s3, dsem):
    c = lax.axis_index("c")
    s = lax.axis_index("s")

    pltpu.sync_copy(znd.at[pl.ds(s * ROWS_ACC, ROWS_ACC)],
                    acc_sp.at[pl.ds(s * ROWS_ACC, ROWS_ACC)])
    pltpu.sync_copy(znp.at[pl.ds(s * (NPAD // NS), NPAD // NS)],
                    deg_sp.at[pl.ds(s * (NPAD // NS), NPAD // NS)])

    plsc.subcore_barrier()

    def _deg_phase():
      for u in range(NC):
        for start, cnt in DEG_BLOCKS:
            pltpu.sync_copy(row_t.at[s, u, pl.ds(start, cnt)],
                            row_blk.at[pl.ds(0, cnt)])
            pltpu.sync_copy(ew_t.at[s, u, pl.ds(start, cnt)],
                            ew_blk.at[pl.ds(0, cnt)])

            @pl.loop(0, cnt)
            def _deg(j):
                pltpu.async_copy(ew_blk.at[j], deg_sp.at[row_blk.at[j]], dsem,
                                 add=True)

            pltpu.make_async_copy(ew_t.at[s, u, pl.ds(start, cnt)],
                                  ew_blk.at[pl.ds(0, cnt)], dsem).wait()

    with jax.named_scope("deg_phase"):
        _deg_phase()
    plsc.subcore_barrier()

    def _dinv_phase():
      pltpu.sync_copy(deg_sp.at[pl.ds(0, N)], dinvT)

      @pl.loop(0, N // L, unroll=4)
      def _rs(k):
          dinvT[pl.ds(k * L, L)] = _rsqrt16(dinvT[pl.ds(k * L, L)])

    with jax.named_scope("dinv_phase"):
        _dinv_phase()
    rows = (rows0, rows1, rows2, rows3)
    gsem = (g0, g1, g2, g3)
    ssem = (s0, s1, s2, s3)
    NBUF = 4

    def _chunk(j, j_local, cnt, p, row_h):
        bufp, gp, sp = rows[p], gsem[p], ssem[p]
        q = (p + 2) % NBUF
        pltpu.make_async_copy(x_hbm.at[pl.ds(0, CH)], bufp, gp).wait()

        @pl.when(j_local > 1)
        def _():
            pltpu.make_async_copy(x_hbm.at[pl.ds(0, CH)], rows[q],
                                  ssem[q]).wait()

        @pl.when(j < CH_MAIN - 2)
        def _():
            pltpu.async_copy(x_hbm.at[row_h.at[j_local + 2]], rows[q],
                             gsem[q])

        @pl.loop(0, CH // L)
        def _scale(g):
            i16 = row_h[j_local, pl.ds(g * L, L)]
            s16 = (plsc.load_gather(dinvT, [i16])
                   * ew_blk[j_local, pl.ds(g * L, L)])
            base = g * L
            for ii in range(L):
                f = s16[ii]
                for qq in range(D // L):
                    bufp[base + ii, pl.ds(qq * L, L)] = (
                        bufp[base + ii, pl.ds(qq * L, L)] * f)

        pltpu.async_copy(bufp, acc_sp.at[col_blk.at[j_local]], sp, add=True)

    def _main_phase():
      row_hs = (row_h0, row_h1)
      for bi, (start, cnt) in enumerate(DEG_BLOCKS):
        row_h = row_hs[bi % 2]
        hcnt = min(cnt + 2, CH_MAIN - start)
        if start > 0:
            for back in (2, 1):
                pp = (start - back) % NBUF
                pltpu.make_async_copy(x_hbm.at[pl.ds(0, CH)], rows[pp],
                                      ssem[pp]).wait()
        pltpu.sync_copy(col_m.at[s, c, pl.ds(start, cnt)],
                        col_blk.at[pl.ds(0, cnt)])
        pltpu.sync_copy(ew_t.at[s, c, pl.ds(start, cnt)],
                        ew_blk.at[pl.ds(0, cnt)])
        pltpu.sync_copy(row_t.at[s, c, pl.ds(start, hcnt)],
                        row_h.at[pl.ds(0, hcnt)])
        if start == 0:
            pltpu.async_copy(x_hbm.at[row_h.at[0]], rows[0], gsem[0])
            pltpu.async_copy(x_hbm.at[row_h.at[1]], rows[1], gsem[1])

        @pl.loop(0, cnt)
        def _main(j_local):
            j = start + j_local

            for pv in range(NBUF):
                @pl.when(j % NBUF == pv)
                def _(pv=pv):
                    _chunk(j, j_local, cnt, pv, row_h)

      for back in (2, 1):
          pp = (CH_MAIN - back) % NBUF
          pltpu.make_async_copy(x_hbm.at[pl.ds(0, CH)], rows[pp],
                                ssem[pp]).wait()

    with jax.named_scope("main_phase"):
        _main_phase()
    plsc.subcore_barrier()

    pltpu.sync_copy(acc_sp.at[pl.ds(s * ROWS_ACC, ROWS_ACC)],
                    acc_out.at[c, pl.ds(s * ROWS_ACC, ROWS_ACC)])

    @pl.when(s == 0)
    def _():
        pltpu.sync_copy(dinvT, dinv_out.at[c])


def _sc_scatter(x, row_t, ew_t, col_m, znd, znp):
    mesh = plsc.VectorSubcoreMesh(core_axis_name="c", subcore_axis_name="s")
    return pl.kernel(
        _sc_body,
        out_type=(
            jax.ShapeDtypeStruct((NC, N, D), jnp.float32),
            jax.ShapeDtypeStruct((NC, N), jnp.float32),
        ),
        mesh=mesh,
        compiler_params=pltpu.CompilerParams(
            needs_layout_passes=False, use_tc_tiling_on_sc=False),
        scratch_types=[
            pltpu.VMEM_SHARED((N, D), jnp.float32),
            pltpu.VMEM_SHARED((NPAD,), jnp.float32),
            pltpu.VMEM((G + 2, CH), jnp.int32),
            pltpu.VMEM((G + 2, CH), jnp.int32),
            pltpu.VMEM((G, CH), jnp.int32),
            pltpu.VMEM((G, CH), jnp.int32),
            pltpu.VMEM((G, CH), jnp.float32),
            pltpu.VMEM((CH, D), jnp.float32),
            pltpu.VMEM((CH, D), jnp.float32),
            pltpu.VMEM((CH, D), jnp.float32),
            pltpu.VMEM((CH, D), jnp.float32),
            pltpu.VMEM((N,), jnp.float32),
            pltpu.SemaphoreType.DMA,
            pltpu.SemaphoreType.DMA,
            pltpu.SemaphoreType.DMA,
            pltpu.SemaphoreType.DMA,
            pltpu.SemaphoreType.DMA,
            pltpu.SemaphoreType.DMA,
            pltpu.SemaphoreType.DMA,
            pltpu.SemaphoreType.DMA,
            pltpu.SemaphoreType.DMA,
        ],
    )(x, row_t, ew_t, col_m, znd, znp)


BT = 1000


def _tc_body(x_ref, a0_ref, a1_ref, dv_ref, A0, A1, C0, C1, bz, bh, o_ref):
    dv = dv_ref[...]
    xb = x_ref[...]
    tx1 = -(dv * (a0_ref[...] + a1_ref[...]))
    z = jax.nn.sigmoid(
        jnp.dot(xb, A0[...], preferred_element_type=jnp.float32)
        + jnp.dot(tx1, A1[...], preferred_element_type=jnp.float32)
        + bz[...])
    ht = jnp.tanh(
        jnp.dot(xb, C0[...], preferred_element_type=jnp.float32)
        + jnp.dot(tx1, C1[...], preferred_element_type=jnp.float32)
        + bh[...])
    o_ref[...] = (1.0 - z) * ht


def _tc_finish(x, a0, a1, dinv, A0, A1, C0, C1, bz, bh):
    full = pl.BlockSpec((D, D), lambda i: (0, 0))
    vec = pl.BlockSpec((1, D), lambda i: (0, 0))
    return pl.pallas_call(
        _tc_body,
        grid=(N // BT,),
        in_specs=[
            pl.BlockSpec((BT, D), lambda i: (i, 0)),
            pl.BlockSpec((BT, D), lambda i: (i, 0)),
            pl.BlockSpec((BT, D), lambda i: (i, 0)),
            pl.BlockSpec((BT, 1), lambda i: (i, 0)),
            full, full, full, full, vec, vec,
        ],
        out_specs=pl.BlockSpec((BT, D), lambda i: (i, 0)),
        out_shape=jax.ShapeDtypeStruct((N, D), jnp.float32),
    )(x, a0, a1, dinv, A0, A1, C0, C1, bz, bh)


def kernel(x, edge_index, edge_weight,
           W_xz, b_xz, W_hz, b_hz, W_xr, b_xr, W_hr, b_hr,
           W_xh, b_xh, W_hh, b_hh):
    row = edge_index[0]
    col = edge_index[1]
    pad_i = jnp.zeros((NW, EPAD - EDGES_MAIN), jnp.int32)
    pad_f = jnp.zeros((NW, EPAD - EDGES_MAIN), jnp.float32)
    row_t = jnp.concatenate([row.reshape(NW, EDGES_MAIN), pad_i], axis=1
                            ).reshape(NS, NC, CH_MAIN, CH)
    ew_t = jnp.concatenate([edge_weight.reshape(NW, EDGES_MAIN), pad_f], axis=1
                           ).reshape(NS, NC, CH_MAIN, CH)
    col_m = jnp.concatenate([col.reshape(NW, EDGES_MAIN), pad_i], axis=1
                            ).reshape(NS, NC, CH_MAIN, CH)
    znd = jnp.zeros((N, D), jnp.float32)
    znp = jnp.zeros((NPAD,), jnp.float32)

    acc, dinv = _sc_scatter(x, row_t, ew_t, col_m, znd, znp)

    return _tc_finish(
        x, acc[0], acc[1], dinv[0].reshape(N, 1),
        W_xz[0], W_xz[1], W_xh[0], W_xh[1],
        (b_xz + b_hz).reshape(1, D), (b_xh + b_hh).reshape(1, D),
    )


# speedup vs baseline: 36.0198x; 1.0806x over previous
import jax
import jax.numpy as jnp
from jax import lax
from jax.experimental import pallas as pl
from jax.experimental.pallas import tpu as pltpu
from jax.experimental.pallas import tpu_sc as plsc

N = 10000
E = 320000
D = 128
NC = 2
NS = 16
NW = NC * NS
L = 16

CH = 64
EDGES_MAIN = E // NW
EPAD = 10048
CH_MAIN = EPAD // CH
NPAD = 10240
ROWS_ACC = N // NS
G = 16
DEG_BLOCKS = [(b * G, min(G, CH_MAIN - b * G))
              for b in range((CH_MAIN + G - 1) // G)]


def _rsqrt16(d16):
    i = lax.bitcast_convert_type(d16, jnp.int32)
    i = jnp.int32(0x5F3759DF) - lax.shift_right_arithmetic(i, 1)
    y = lax.bitcast_convert_type(i, jnp.float32)
    half = d16 * 0.5
    for _ in range(3):
        y = y * (1.5 - half * y * y)
    return jnp.where(d16 > 0.0, y, 0.0)


def _sc_body(x_hbm, row_t, ew_t, col_m, znd, znp,
             acc_out, dinv_out,
             acc_sp, deg_sp, row_h0, row_h1, col_blk, row_blk, ew_blk,
             row_blk2, ew_blk2, sg0, sg1,
             rows0, rows1, rows2, rows3, dinvT, g0, g1, g2, g3,
             s0, s1, s2, s3, dsem):
    c = lax.axis_index("c")
    s = lax.axis_index("s")

    pltpu.sync_copy(znd.at[pl.ds(s * ROWS_ACC, ROWS_ACC)],
                    acc_sp.at[pl.ds(s * ROWS_ACC, ROWS_ACC)])
    pltpu.sync_copy(znp.at[pl.ds(s * (NPAD // NS), NPAD // NS)],
                    deg_sp.at[pl.ds(s * (NPAD // NS), NPAD // NS)])

    plsc.subcore_barrier()

    def _deg_phase():
      blocks = [(u, start, cnt) for u in range(NC)
                for start, cnt in DEG_BLOCKS]
      rblk = (row_blk, row_blk2)
      eblk = (ew_blk, ew_blk2)
      stg = (sg0, sg1)
      u0, st0, cn0 = blocks[0]
      pltpu.async_copy(row_t.at[s, u0, pl.ds(st0, cn0)],
                       rblk[0].at[pl.ds(0, cn0)], stg[0])
      pltpu.async_copy(ew_t.at[s, u0, pl.ds(st0, cn0)],
                       eblk[0].at[pl.ds(0, cn0)], stg[0])
      for m, (u, start, cnt) in enumerate(blocks):
          pr = m % 2
          pltpu.make_async_copy(row_t.at[s, u, pl.ds(start, cnt)],
                                rblk[pr].at[pl.ds(0, cnt)], stg[pr]).wait()
          pltpu.make_async_copy(ew_t.at[s, u, pl.ds(start, cnt)],
                                eblk[pr].at[pl.ds(0, cnt)], stg[pr]).wait()
          if m + 1 < len(blocks):
              if m >= 1:
                  up, sp_, cp = blocks[m - 1]
                  pltpu.make_async_copy(ew_t.at[s, up, pl.ds(sp_, cp)],
                                        eblk[1 - pr].at[pl.ds(0, cp)],
                                        dsem).wait()
              un, sn, cn = blocks[m + 1]
              pltpu.async_copy(row_t.at[s, un, pl.ds(sn, cn)],
                               rblk[1 - pr].at[pl.ds(0, cn)], stg[1 - pr])
              pltpu.async_copy(ew_t.at[s, un, pl.ds(sn, cn)],
                               eblk[1 - pr].at[pl.ds(0, cn)], stg[1 - pr])

          @pl.loop(0, cnt)
          def _deg(j):
              pltpu.async_copy(eblk[pr].at[j], deg_sp.at[rblk[pr].at[j]],
                               dsem, add=True)

      ul, sl, cl = blocks[-1]
      pltpu.make_async_copy(ew_t.at[s, ul, pl.ds(sl, cl)],
                            eblk[len(blocks[:-1]) % 2].at[pl.ds(0, cl)],
                            dsem).wait()
      um, sm, cm = blocks[-2]
      pltpu.make_async_copy(ew_t.at[s, um, pl.ds(sm, cm)],
                            eblk[(len(blocks) - 2) % 2].at[pl.ds(0, cm)],
                            dsem).wait()

    with jax.named_scope("deg_phase"):
        _deg_phase()
    plsc.subcore_barrier()

    def _dinv_phase():
      pltpu.sync_copy(deg_sp.at[pl.ds(0, N)], dinvT)

      @pl.loop(0, N // L, unroll=4)
      def _rs(k):
          dinvT[pl.ds(k * L, L)] = _rsqrt16(dinvT[pl.ds(k * L, L)])

    with jax.named_scope("dinv_phase"):
        _dinv_phase()
    rows = (rows0, rows1, rows2, rows3)
    gsem = (g0, g1, g2, g3)
    ssem = (s0, s1, s2, s3)
    NBUF = 4

    def _chunk(j, j_local, cnt, p, row_h):
        bufp, gp, sp = rows[p], gsem[p], ssem[p]
        q = (p + 2) % NBUF
        pltpu.make_async_copy(x_hbm.at[pl.ds(0, CH)], bufp, gp).wait()

        @pl.when(j_local > 1)
        def _():
            pltpu.make_async_copy(x_hbm.at[pl.ds(0, CH)], rows[q],
                                  ssem[q]).wait()

        @pl.when(j < CH_MAIN - 2)
        def _():
            pltpu.async_copy(x_hbm.at[row_h.at[j_local + 2]], rows[q],
                             gsem[q])

        @pl.loop(0, CH // L)
        def _scale(g):
            i16 = row_h[j_local, pl.ds(g * L, L)]
            s16 = (plsc.load_gather(dinvT, [i16])
                   * ew_blk[j_local, pl.ds(g * L, L)])
            base = g * L
            for ii in range(L):
                f = s16[ii]
                for qq in range(D // L):
                    bufp[base + ii, pl.ds(qq * L, L)] = (
                        bufp[base + ii, pl.ds(qq * L, L)] * f)

        pltpu.async_copy(bufp, acc_sp.at[col_blk.at[j_local]], sp, add=True)

    def _main_phase():
      row_hs = (row_h0, row_h1)
      for bi, (start, cnt) in enumerate(DEG_BLOCKS):
        row_h = row_hs[bi % 2]
        hcnt = min(cnt + 2, CH_MAIN - start)
        if start > 0:
            for back in (2, 1):
                pp = (start - back) % NBUF
                pltpu.make_async_copy(x_hbm.at[pl.ds(0, CH)], rows[pp],
                                      ssem[pp]).wait()
        pltpu.sync_copy(col_m.at[s, c, pl.ds(start, cnt)],
                        col_blk.at[pl.ds(0, cnt)])
        pltpu.sync_copy(ew_t.at[s, c, pl.ds(start, cnt)],
                        ew_blk.at[pl.ds(0, cnt)])
        pltpu.sync_copy(row_t.at[s, c, pl.ds(start, hcnt)],
                        row_h.at[pl.ds(0, hcnt)])
        if start == 0:
            pltpu.async_copy(x_hbm.at[row_h.at[0]], rows[0], gsem[0])
            pltpu.async_copy(x_hbm.at[row_h.at[1]], rows[1], gsem[1])

        @pl.loop(0, cnt)
        def _main(j_local):
            j = start + j_local

            for pv in range(NBUF):
                @pl.when(j % NBUF == pv)
                def _(pv=pv):
                    _chunk(j, j_local, cnt, pv, row_h)

      for back in (2, 1):
          pp = (CH_MAIN - back) % NBUF
          pltpu.make_async_copy(x_hbm.at[pl.ds(0, CH)], rows[pp],
                                ssem[pp]).wait()

    with jax.named_scope("main_phase"):
        _main_phase()
    plsc.subcore_barrier()

    pltpu.sync_copy(acc_sp.at[pl.ds(s * ROWS_ACC, ROWS_ACC)],
                    acc_out.at[c, pl.ds(s * ROWS_ACC, ROWS_ACC)])

    @pl.when(s == 0)
    def _():
        pltpu.sync_copy(dinvT, dinv_out.at[c])


def _sc_scatter(x, row_t, ew_t, col_m, znd, znp):
    mesh = plsc.VectorSubcoreMesh(core_axis_name="c", subcore_axis_name="s")
    return pl.kernel(
        _sc_body,
        out_type=(
            jax.ShapeDtypeStruct((NC, N, D), jnp.float32),
            jax.ShapeDtypeStruct((NC, N), jnp.float32),
        ),
        mesh=mesh,
        compiler_params=pltpu.CompilerParams(
            needs_layout_passes=False, use_tc_tiling_on_sc=False),
        scratch_types=[
            pltpu.VMEM_SHARED((N, D), jnp.float32),
            pltpu.VMEM_SHARED((NPAD,), jnp.float32),
            pltpu.VMEM((G + 2, CH), jnp.int32),
            pltpu.VMEM((G + 2, CH), jnp.int32),
            pltpu.VMEM((G, CH), jnp.int32),
            pltpu.VMEM((G, CH), jnp.int32),
            pltpu.VMEM((G, CH), jnp.float32),
            pltpu.VMEM((G, CH), jnp.int32),
            pltpu.VMEM((G, CH), jnp.float32),
            pltpu.SemaphoreType.DMA,
            pltpu.SemaphoreType.DMA,
            pltpu.VMEM((CH, D), jnp.float32),
            pltpu.VMEM((CH, D), jnp.float32),
            pltpu.VMEM((CH, D), jnp.float32),
            pltpu.VMEM((CH, D), jnp.float32),
            pltpu.VMEM((N,), jnp.float32),
            pltpu.SemaphoreType.DMA,
            pltpu.SemaphoreType.DMA,
            pltpu.SemaphoreType.DMA,
            pltpu.SemaphoreType.DMA,
            pltpu.SemaphoreType.DMA,
            pltpu.SemaphoreType.DMA,
            pltpu.SemaphoreType.DMA,
            pltpu.SemaphoreType.DMA,
            pltpu.SemaphoreType.DMA,
        ],
    )(x, row_t, ew_t, col_m, znd, znp)


BT = 1000


def _tc_body(x_ref, acc_ref, dv_ref, A0, A1, C0, C1, bz, bh, o_ref):
    dv = dv_ref[...]
    xb = x_ref[...]
    tx1 = -(dv * (acc_ref[0] + acc_ref[1]))
    z = jax.nn.sigmoid(
        jnp.dot(xb, A0[...], preferred_element_type=jnp.float32)
        + jnp.dot(tx1, A1[...], preferred_element_type=jnp.float32)
        + bz[...])
    ht = jnp.tanh(
        jnp.dot(xb, C0[...], preferred_element_type=jnp.float32)
        + jnp.dot(tx1, C1[...], preferred_element_type=jnp.float32)
        + bh[...])
    o_ref[...] = (1.0 - z) * ht


def _tc_finish(x, acc, dinv, A0, A1, C0, C1, bz, bh):
    full = pl.BlockSpec((D, D), lambda i: (0, 0))
    vec = pl.BlockSpec((1, D), lambda i: (0, 0))
    return pl.pallas_call(
        _tc_body,
        grid=(N // BT,),
        in_specs=[
            pl.BlockSpec((BT, D), lambda i: (i, 0)),
            pl.BlockSpec((NC, BT, D), lambda i: (0, i, 0)),
            pl.BlockSpec((BT, 1), lambda i: (i, 0)),
            full, full, full, full, vec, vec,
        ],
        out_specs=pl.BlockSpec((BT, D), lambda i: (i, 0)),
        out_shape=jax.ShapeDtypeStruct((N, D), jnp.float32),
    )(x, acc, dinv, A0, A1, C0, C1, bz, bh)


def kernel(x, edge_index, edge_weight,
           W_xz, b_xz, W_hz, b_hz, W_xr, b_xr, W_hr, b_hr,
           W_xh, b_xh, W_hh, b_hh):
    row = edge_index[0]
    col = edge_index[1]
    pad_i = jnp.zeros((NW, EPAD - EDGES_MAIN), jnp.int32)
    pad_f = jnp.zeros((NW, EPAD - EDGES_MAIN), jnp.float32)
    row_t = jnp.concatenate([row.reshape(NW, EDGES_MAIN), pad_i], axis=1
                            ).reshape(NS, NC, CH_MAIN, CH)
    ew_t = jnp.concatenate([edge_weight.reshape(NW, EDGES_MAIN), pad_f], axis=1
                           ).reshape(NS, NC, CH_MAIN, CH)
    col_m = jnp.concatenate([col.reshape(NW, EDGES_MAIN), pad_i], axis=1
                            ).reshape(NS, NC, CH_MAIN, CH)
    znd = jnp.zeros((N, D), jnp.float32)
    znp = jnp.zeros((NPAD,), jnp.float32)

    acc, dinv = _sc_scatter(x, row_t, ew_t, col_m, znd, znp)

    return _tc_finish(
        x, acc, dinv[0].reshape(N, 1),
        W_xz[0], W_xz[1], W_xh[0], W_xh[1],
        (b_xz + b_hz).reshape(1, D), (b_xh + b_hh).reshape(1, D),
    )
